# baseline (speedup 1.0000x reference)
import functools

import jax
import jax.numpy as jnp
from jax import lax
from jax.experimental import pallas as pl
from jax.experimental.pallas import tpu as pltpu
from jax.experimental.pallas import tpu_sc as plsc

B = 16384
D = 50
H = 128
NU = 1000000
NM = 100000
NC = 2
NS = 16
NW = NC * NS
BPW = B // NW
STAGE = 128
GRP = 16
NGRP = STAGE // GRP
PBLK = 2048


def _bf16_bits(b):
    return ((b + 0x7FFF + ((b >> 16) & 1)) >> 16) & 0xFFFF


def _proj_body(tT_ref, w_ref, out_ref):
    h = lax.dot_general(tT_ref[...], w_ref[...], (((0,), (0,)), ((), ())),
                        preferred_element_type=jnp.float32)
    bits = lax.bitcast_convert_type(h, jnp.int32)
    lo = _bf16_bits(bits[:, :H // 2])
    hi = _bf16_bits(bits[:, H // 2:])
    out_ref[...] = lax.bitcast_convert_type(lo | (hi << 16), jnp.float32)


def _tc_proj(tT, w, n):
    nblk = (n + PBLK - 1) // PBLK
    return pl.pallas_call(
        _proj_body,
        grid=(nblk,),
        in_specs=[
            pl.BlockSpec((D, PBLK), lambda i: (0, i)),
            pl.BlockSpec((D, H), lambda i: (0, 0)),
        ],
        out_specs=pl.BlockSpec((PBLK, H // 2), lambda i: (i, 0)),
        out_shape=jax.ShapeDtypeStruct((n, H // 2), jnp.float32),
    )(tT, w)


def _gather_body(uidx_hbm, midx_hbm, ut_hbm, mt_hbm, ue_hbm, me_hbm,
                 uidx_v, midx_v, urows_v, mrows_v, sem):
    wid = lax.axis_index("s") * NC + lax.axis_index("c")
    base = wid * BPW
    pltpu.sync_copy(uidx_hbm.at[pl.ds(base, BPW)], uidx_v)
    pltpu.sync_copy(midx_hbm.at[pl.ds(base, BPW)], midx_v)

    for st in range(BPW // STAGE):
        st0 = st * STAGE

        def fire(g, slot):
            uv = uidx_v[pl.ds(st0 + g * GRP, GRP)]
            mv = midx_v[pl.ds(st0 + g * GRP, GRP)]
            for j in range(GRP):
                r = g * GRP + j
                pltpu.make_async_copy(
                    ut_hbm.at[uv[j]], urows_v.at[r],
                    sem.at[slot]).start()
                pltpu.make_async_copy(
                    mt_hbm.at[mv[j]], mrows_v.at[r],
                    sem.at[slot]).start()

        def drain(slot):
            for j in range(GRP):
                pltpu.make_async_copy(
                    ut_hbm.at[0], urows_v.at[0], sem.at[slot]).wait()
                pltpu.make_async_copy(
                    mt_hbm.at[0], mrows_v.at[0], sem.at[slot]).wait()

        fire(0, 0)
        fire(1, 1)

        def stepg(g, carry):
            sb = lax.rem(g, 2)
            drain(sb)

            @pl.when(g + 2 < NGRP)
            def _():
                fire(g + 2, sb)

            return carry

        lax.fori_loop(0, NGRP, stepg, 0)
        pltpu.sync_copy(urows_v, ue_hbm.at[pl.ds(base + st0, STAGE)])
        pltpu.sync_copy(mrows_v, me_hbm.at[pl.ds(base + st0, STAGE)])


def _sc_gather(uidx, midx, ut3, mt3):
    mesh = plsc.VectorSubcoreMesh(core_axis_name="c", subcore_axis_name="s")
    fn = pl.kernel(
        _gather_body,
        out_type=[
            jax.ShapeDtypeStruct((B, H // 2), jnp.float32),
            jax.ShapeDtypeStruct((B, H // 2), jnp.float32),
        ],
        mesh=mesh,
        scratch_types=[
            pltpu.VMEM((BPW,), jnp.int32),
            pltpu.VMEM((BPW,), jnp.int32),
            pltpu.VMEM((STAGE, H // 2), jnp.float32),
            pltpu.VMEM((STAGE, H // 2), jnp.float32),
            pltpu.SemaphoreType.DMA((2,)),
        ],
    )
    return fn(uidx, midx, ut3, mt3)


def _unpack(packed):
    vi = lax.bitcast_convert_type(packed, jnp.int32)
    lo = lax.bitcast_convert_type(vi << 16, jnp.float32)
    hi = lax.bitcast_convert_type(
        vi & jnp.int32(-65536), jnp.float32)
    return jnp.concatenate([lo, hi], axis=1)


def _mlp_body(hu_ref, hm_ref, b1_ref, w2_ref, b2_ref, out_ref):
    h = _unpack(hu_ref[...]) + _unpack(hm_ref[...]) + b1_ref[...]
    h = jnp.maximum(h, 0.0)
    out_ref[...] = jnp.sum(h * w2_ref[...], axis=1, keepdims=True) + b2_ref[...]


def _tc_mlp(hu, hm, b1, W2, b2):
    b1r = b1.reshape(1, H)
    w2r = W2.reshape(1, H)
    b2r = b2.reshape(1, 1)
    nblk = 8
    bb = B // nblk
    out = pl.pallas_call(
        _mlp_body,
        grid=(nblk,),
        in_specs=[
            pl.BlockSpec((bb, H // 2), lambda i: (i, 0)),
            pl.BlockSpec((bb, H // 2), lambda i: (i, 0)),
            pl.BlockSpec((1, H), lambda i: (0, 0)),
            pl.BlockSpec((1, H), lambda i: (0, 0)),
            pl.BlockSpec((1, 1), lambda i: (0, 0)),
        ],
        out_specs=pl.BlockSpec((bb, 1), lambda i: (i, 0)),
        out_shape=jax.ShapeDtypeStruct((B, 1), jnp.float32),
    )(hu, hm, b1r, w2r, b2r)
    return out[:, 0]


def kernel(user, movie, user_table, movie_table, W1, b1, W2, b2):
    uidx = user.astype(jnp.int32)
    midx = movie.astype(jnp.int32)
    hu_full = _tc_proj(user_table.T, W1[:D], NU)
    hm_full = _tc_proj(movie_table.T, W1[D:], NM)
    hu_g, hm_g = _sc_gather(uidx, midx, hu_full, hm_full)
    return _tc_mlp(hu_g, hm_g, b1, W2, b2)


# speedup vs baseline: 1.3703x; 1.3703x over previous
import functools

import jax
import jax.numpy as jnp
from jax import lax
from jax.experimental import pallas as pl
from jax.experimental.pallas import tpu as pltpu
from jax.experimental.pallas import tpu_sc as plsc

B = 16384
D = 50
H = 128
NU = 1000000
NM = 100000
NC = 2
NS = 16
NW = NC * NS
BPW = B // NW
STAGE = 128
GRP = 16
NGRP = STAGE // GRP
PBLK = 4096


def _proj_body(tT_ref, w_ref, out_ref):
    h = lax.dot_general(tT_ref[...], w_ref[...], (((0,), (0,)), ((), ())),
                        preferred_element_type=jnp.float32)
    bits = lax.bitcast_convert_type(h, jnp.int32) + 0x8000
    lo = ((bits[:, :H // 2]) >> 16) & 0xFFFF
    hi = bits[:, H // 2:] & jnp.int32(-65536)
    out_ref[...] = lax.bitcast_convert_type(lo | hi, jnp.float32)


def _tc_proj(tT, w, n):
    nblk = (n + PBLK - 1) // PBLK
    return pl.pallas_call(
        _proj_body,
        grid=(nblk,),
        in_specs=[
            pl.BlockSpec((D, PBLK), lambda i: (0, i)),
            pl.BlockSpec((D, H), lambda i: (0, 0)),
        ],
        out_specs=pl.BlockSpec((PBLK, H // 2), lambda i: (i, 0)),
        out_shape=jax.ShapeDtypeStruct((n, H // 2), jnp.float32),
    )(tT, w)


def _gather_body(uidx_hbm, midx_hbm, ut_hbm, mt_hbm, ue_hbm, me_hbm,
                 uidx_v, midx_v, urows_v, mrows_v, sem):
    wid = lax.axis_index("s") * NC + lax.axis_index("c")
    base = wid * BPW
    pltpu.sync_copy(uidx_hbm.at[pl.ds(base, BPW)], uidx_v)
    pltpu.sync_copy(midx_hbm.at[pl.ds(base, BPW)], midx_v)

    for st in range(BPW // STAGE):
        st0 = st * STAGE

        def fire(g, slot):
            uv = uidx_v[pl.ds(st0 + g * GRP, GRP)]
            mv = midx_v[pl.ds(st0 + g * GRP, GRP)]
            for j in range(GRP):
                r = g * GRP + j
                pltpu.make_async_copy(
                    ut_hbm.at[uv[j]], urows_v.at[r],
                    sem.at[slot]).start()
                pltpu.make_async_copy(
                    mt_hbm.at[mv[j]], mrows_v.at[r],
                    sem.at[slot]).start()

        def drain(slot):
            for j in range(GRP):
                pltpu.make_async_copy(
                    ut_hbm.at[0], urows_v.at[0], sem.at[slot]).wait()
                pltpu.make_async_copy(
                    mt_hbm.at[0], mrows_v.at[0], sem.at[slot]).wait()

        fire(0, 0)
        fire(1, 1)

        def stepg(g, carry):
            sb = lax.rem(g, 2)
            drain(sb)

            @pl.when(g + 2 < NGRP)
            def _():
                fire(g + 2, sb)

            return carry

        lax.fori_loop(0, NGRP, stepg, 0)
        pltpu.sync_copy(urows_v, ue_hbm.at[pl.ds(base + st0, STAGE)])
        pltpu.sync_copy(mrows_v, me_hbm.at[pl.ds(base + st0, STAGE)])


def _sc_gather(uidx, midx, ut3, mt3):
    mesh = plsc.VectorSubcoreMesh(core_axis_name="c", subcore_axis_name="s")
    fn = pl.kernel(
        _gather_body,
        out_type=[
            jax.ShapeDtypeStruct((B, H // 2), jnp.float32),
            jax.ShapeDtypeStruct((B, H // 2), jnp.float32),
        ],
        mesh=mesh,
        scratch_types=[
            pltpu.VMEM((BPW,), jnp.int32),
            pltpu.VMEM((BPW,), jnp.int32),
            pltpu.VMEM((STAGE, H // 2), jnp.float32),
            pltpu.VMEM((STAGE, H // 2), jnp.float32),
            pltpu.SemaphoreType.DMA((2,)),
        ],
    )
    return fn(uidx, midx, ut3, mt3)


def _unpack(packed):
    vi = lax.bitcast_convert_type(packed, jnp.int32)
    lo = lax.bitcast_convert_type(vi << 16, jnp.float32)
    hi = lax.bitcast_convert_type(
        vi & jnp.int32(-65536), jnp.float32)
    return jnp.concatenate([lo, hi], axis=1)


def _mlp_body(hu_ref, hm_ref, b1_ref, w2_ref, b2_ref, out_ref):
    h = _unpack(hu_ref[...]) + _unpack(hm_ref[...]) + b1_ref[...]
    h = jnp.maximum(h, 0.0)
    out_ref[...] = jnp.sum(h * w2_ref[...], axis=1, keepdims=True) + b2_ref[...]


def _tc_mlp(hu, hm, b1, W2, b2):
    b1r = b1.reshape(1, H)
    w2r = W2.reshape(1, H)
    b2r = b2.reshape(1, 1)
    nblk = 8
    bb = B // nblk
    out = pl.pallas_call(
        _mlp_body,
        grid=(nblk,),
        in_specs=[
            pl.BlockSpec((bb, H // 2), lambda i: (i, 0)),
            pl.BlockSpec((bb, H // 2), lambda i: (i, 0)),
            pl.BlockSpec((1, H), lambda i: (0, 0)),
            pl.BlockSpec((1, H), lambda i: (0, 0)),
            pl.BlockSpec((1, 1), lambda i: (0, 0)),
        ],
        out_specs=pl.BlockSpec((bb, 1), lambda i: (i, 0)),
        out_shape=jax.ShapeDtypeStruct((B, 1), jnp.float32),
    )(hu, hm, b1r, w2r, b2r)
    return out[:, 0]


def kernel(user, movie, user_table, movie_table, W1, b1, W2, b2):
    uidx = user.astype(jnp.int32)
    midx = movie.astype(jnp.int32)
    hu_full = _tc_proj(user_table.T, W1[:D], NU)
    hm_full = _tc_proj(movie_table.T, W1[D:], NM)
    hu_g, hm_g = _sc_gather(uidx, midx, hu_full, hm_full)
    return _tc_mlp(hu_g, hm_g, b1, W2, b2)


# speedup vs baseline: 1.6444x; 1.2000x over previous
import functools

import jax
import jax.numpy as jnp
from jax import lax
from jax.experimental import pallas as pl
from jax.experimental.pallas import tpu as pltpu
from jax.experimental.pallas import tpu_sc as plsc

B = 16384
D = 50
H = 128
NU = 1000000
NM = 100000
NC = 2
NS = 16
NW = NC * NS
BPW = B // NW
STAGE = 128
GRP = 16
NGRP = STAGE // GRP
PBLK = 8192


def _proj_body(tT_ref, w_ref, out_ref):
    h = lax.dot_general(tT_ref[...], w_ref[...], (((0,), (0,)), ((), ())),
                        preferred_element_type=jnp.float32)
    bits = lax.bitcast_convert_type(h, jnp.int32) + 0x8000
    lo = ((bits[:, :H // 2]) >> 16) & 0xFFFF
    hi = bits[:, H // 2:] & jnp.int32(-65536)
    out_ref[...] = lax.bitcast_convert_type(lo | hi, jnp.float32)


def _tc_proj(tT, w, n):
    nblk = (n + PBLK - 1) // PBLK
    return pl.pallas_call(
        _proj_body,
        grid=(nblk,),
        in_specs=[
            pl.BlockSpec((D, PBLK), lambda i: (0, i)),
            pl.BlockSpec((D, H), lambda i: (0, 0)),
        ],
        out_specs=pl.BlockSpec((PBLK, H // 2), lambda i: (i, 0)),
        out_shape=jax.ShapeDtypeStruct((n, H // 2), jnp.float32),
    )(tT, w)


def _gather_body(uidx_hbm, midx_hbm, ut_hbm, mt_hbm, ue_hbm, me_hbm,
                 uidx_v, midx_v, urows_v, mrows_v, sem):
    wid = lax.axis_index("s") * NC + lax.axis_index("c")
    base = wid * BPW
    pltpu.sync_copy(uidx_hbm.at[pl.ds(base, BPW)], uidx_v)
    pltpu.sync_copy(midx_hbm.at[pl.ds(base, BPW)], midx_v)

    for st in range(BPW // STAGE):
        st0 = st * STAGE

        def fire(g, slot):
            uv = uidx_v[pl.ds(st0 + g * GRP, GRP)]
            mv = midx_v[pl.ds(st0 + g * GRP, GRP)]
            for j in range(GRP):
                r = g * GRP + j
                pltpu.make_async_copy(
                    ut_hbm.at[uv[j]], urows_v.at[r],
                    sem.at[slot]).start()
                pltpu.make_async_copy(
                    mt_hbm.at[mv[j]], mrows_v.at[r],
                    sem.at[slot]).start()

        def drain(slot):
            for j in range(GRP):
                pltpu.make_async_copy(
                    ut_hbm.at[0], urows_v.at[0], sem.at[slot]).wait()
                pltpu.make_async_copy(
                    mt_hbm.at[0], mrows_v.at[0], sem.at[slot]).wait()

        fire(0, 0)
        fire(1, 1)

        def stepg(g, carry):
            sb = lax.rem(g, 2)
            drain(sb)

            @pl.when(g + 2 < NGRP)
            def _():
                fire(g + 2, sb)

            return carry

        lax.fori_loop(0, NGRP, stepg, 0)
        pltpu.sync_copy(urows_v, ue_hbm.at[pl.ds(base + st0, STAGE)])
        pltpu.sync_copy(mrows_v, me_hbm.at[pl.ds(base + st0, STAGE)])


def _sc_gather(uidx, midx, ut3, mt3):
    mesh = plsc.VectorSubcoreMesh(core_axis_name="c", subcore_axis_name="s")
    fn = pl.kernel(
        _gather_body,
        out_type=[
            jax.ShapeDtypeStruct((B, H // 2), jnp.float32),
            jax.ShapeDtypeStruct((B, H // 2), jnp.float32),
        ],
        mesh=mesh,
        scratch_types=[
            pltpu.VMEM((BPW,), jnp.int32),
            pltpu.VMEM((BPW,), jnp.int32),
            pltpu.VMEM((STAGE, H // 2), jnp.float32),
            pltpu.VMEM((STAGE, H // 2), jnp.float32),
            pltpu.SemaphoreType.DMA((2,)),
        ],
    )
    return fn(uidx, midx, ut3, mt3)


def _unpack(packed):
    vi = lax.bitcast_convert_type(packed, jnp.int32)
    lo = lax.bitcast_convert_type(vi << 16, jnp.float32)
    hi = lax.bitcast_convert_type(
        vi & jnp.int32(-65536), jnp.float32)
    return jnp.concatenate([lo, hi], axis=1)


def _mlp_body(hu_ref, hm_ref, b1_ref, w2_ref, b2_ref, out_ref):
    h = _unpack(hu_ref[...]) + _unpack(hm_ref[...]) + b1_ref[...]
    h = jnp.maximum(h, 0.0)
    out_ref[...] = jnp.sum(h * w2_ref[...], axis=1, keepdims=True) + b2_ref[...]


def _tc_mlp(hu, hm, b1, W2, b2):
    b1r = b1.reshape(1, H)
    w2r = W2.reshape(1, H)
    b2r = b2.reshape(1, 1)
    nblk = 8
    bb = B // nblk
    out = pl.pallas_call(
        _mlp_body,
        grid=(nblk,),
        in_specs=[
            pl.BlockSpec((bb, H // 2), lambda i: (i, 0)),
            pl.BlockSpec((bb, H // 2), lambda i: (i, 0)),
            pl.BlockSpec((1, H), lambda i: (0, 0)),
            pl.BlockSpec((1, H), lambda i: (0, 0)),
            pl.BlockSpec((1, 1), lambda i: (0, 0)),
        ],
        out_specs=pl.BlockSpec((bb, 1), lambda i: (i, 0)),
        out_shape=jax.ShapeDtypeStruct((B, 1), jnp.float32),
    )(hu, hm, b1r, w2r, b2r)
    return out[:, 0]


def kernel(user, movie, user_table, movie_table, W1, b1, W2, b2):
    uidx = user.astype(jnp.int32)
    midx = movie.astype(jnp.int32)
    hu_full = _tc_proj(user_table.T, W1[:D], NU)
    hm_full = _tc_proj(movie_table.T, W1[D:], NM)
    hu_g, hm_g = _sc_gather(uidx, midx, hu_full, hm_full)
    return _tc_mlp(hu_g, hm_g, b1, W2, b2)


# speedup vs baseline: 1.8384x; 1.1180x over previous
import functools

import jax
import jax.numpy as jnp
from jax import lax
from jax.experimental import pallas as pl
from jax.experimental.pallas import tpu as pltpu
from jax.experimental.pallas import tpu_sc as plsc

B = 16384
D = 50
H = 128
NU = 1000000
NM = 100000
NC = 2
NS = 16
NW = NC * NS
BPW = B // NW
STAGE = 128
GRP = 16
NGRP = STAGE // GRP
PBLK = 16384


def _proj_body(tT_ref, w_ref, out_ref):
    h = lax.dot_general(tT_ref[...], w_ref[...], (((0,), (0,)), ((), ())),
                        preferred_element_type=jnp.float32)
    bits = lax.bitcast_convert_type(h, jnp.int32) + 0x8000
    lo = ((bits[:, :H // 2]) >> 16) & 0xFFFF
    hi = bits[:, H // 2:] & jnp.int32(-65536)
    out_ref[...] = lax.bitcast_convert_type(lo | hi, jnp.float32)


def _tc_proj(tT, w, n):
    nblk = (n + PBLK - 1) // PBLK
    return pl.pallas_call(
        _proj_body,
        grid=(nblk,),
        in_specs=[
            pl.BlockSpec((D, PBLK), lambda i: (0, i)),
            pl.BlockSpec((D, H), lambda i: (0, 0)),
        ],
        out_specs=pl.BlockSpec((PBLK, H // 2), lambda i: (i, 0)),
        out_shape=jax.ShapeDtypeStruct((n, H // 2), jnp.float32),
    )(tT, w)


def _gather_body(uidx_hbm, midx_hbm, ut_hbm, mt_hbm, ue_hbm, me_hbm,
                 uidx_v, midx_v, urows_v, mrows_v, sem):
    wid = lax.axis_index("s") * NC + lax.axis_index("c")
    base = wid * BPW
    pltpu.sync_copy(uidx_hbm.at[pl.ds(base, BPW)], uidx_v)
    pltpu.sync_copy(midx_hbm.at[pl.ds(base, BPW)], midx_v)

    for st in range(BPW // STAGE):
        st0 = st * STAGE

        def fire(g, slot):
            uv = uidx_v[pl.ds(st0 + g * GRP, GRP)]
            mv = midx_v[pl.ds(st0 + g * GRP, GRP)]
            for j in range(GRP):
                r = g * GRP + j
                pltpu.make_async_copy(
                    ut_hbm.at[uv[j]], urows_v.at[r],
                    sem.at[slot]).start()
                pltpu.make_async_copy(
                    mt_hbm.at[mv[j]], mrows_v.at[r],
                    sem.at[slot]).start()

        def drain(slot):
            for j in range(GRP):
                pltpu.make_async_copy(
                    ut_hbm.at[0], urows_v.at[0], sem.at[slot]).wait()
                pltpu.make_async_copy(
                    mt_hbm.at[0], mrows_v.at[0], sem.at[slot]).wait()

        fire(0, 0)
        fire(1, 1)

        def stepg(g, carry):
            sb = lax.rem(g, 2)
            drain(sb)

            @pl.when(g + 2 < NGRP)
            def _():
                fire(g + 2, sb)

            return carry

        lax.fori_loop(0, NGRP, stepg, 0)
        pltpu.sync_copy(urows_v, ue_hbm.at[pl.ds(base + st0, STAGE)])
        pltpu.sync_copy(mrows_v, me_hbm.at[pl.ds(base + st0, STAGE)])


def _sc_gather(uidx, midx, ut3, mt3):
    mesh = plsc.VectorSubcoreMesh(core_axis_name="c", subcore_axis_name="s")
    fn = pl.kernel(
        _gather_body,
        out_type=[
            jax.ShapeDtypeStruct((B, H // 2), jnp.float32),
            jax.ShapeDtypeStruct((B, H // 2), jnp.float32),
        ],
        mesh=mesh,
        scratch_types=[
            pltpu.VMEM((BPW,), jnp.int32),
            pltpu.VMEM((BPW,), jnp.int32),
            pltpu.VMEM((STAGE, H // 2), jnp.float32),
            pltpu.VMEM((STAGE, H // 2), jnp.float32),
            pltpu.SemaphoreType.DMA((2,)),
        ],
    )
    return fn(uidx, midx, ut3, mt3)


def _unpack(packed):
    vi = lax.bitcast_convert_type(packed, jnp.int32)
    lo = lax.bitcast_convert_type(vi << 16, jnp.float32)
    hi = lax.bitcast_convert_type(
        vi & jnp.int32(-65536), jnp.float32)
    return jnp.concatenate([lo, hi], axis=1)


def _mlp_body(hu_ref, hm_ref, b1_ref, w2_ref, b2_ref, out_ref):
    h = _unpack(hu_ref[...]) + _unpack(hm_ref[...]) + b1_ref[...]
    h = jnp.maximum(h, 0.0)
    out_ref[...] = jnp.sum(h * w2_ref[...], axis=1, keepdims=True) + b2_ref[...]


def _tc_mlp(hu, hm, b1, W2, b2):
    b1r = b1.reshape(1, H)
    w2r = W2.reshape(1, H)
    b2r = b2.reshape(1, 1)
    nblk = 8
    bb = B // nblk
    out = pl.pallas_call(
        _mlp_body,
        grid=(nblk,),
        in_specs=[
            pl.BlockSpec((bb, H // 2), lambda i: (i, 0)),
            pl.BlockSpec((bb, H // 2), lambda i: (i, 0)),
            pl.BlockSpec((1, H), lambda i: (0, 0)),
            pl.BlockSpec((1, H), lambda i: (0, 0)),
            pl.BlockSpec((1, 1), lambda i: (0, 0)),
        ],
        out_specs=pl.BlockSpec((bb, 1), lambda i: (i, 0)),
        out_shape=jax.ShapeDtypeStruct((B, 1), jnp.float32),
    )(hu, hm, b1r, w2r, b2r)
    return out[:, 0]


def kernel(user, movie, user_table, movie_table, W1, b1, W2, b2):
    uidx = user.astype(jnp.int32)
    midx = movie.astype(jnp.int32)
    hu_full = _tc_proj(user_table.T, W1[:D], NU)
    hm_full = _tc_proj(movie_table.T, W1[D:], NM)
    hu_g, hm_g = _sc_gather(uidx, midx, hu_full, hm_full)
    return _tc_mlp(hu_g, hm_g, b1, W2, b2)
